# baseline (speedup 1.0000x reference)
import jax
import jax.numpy as jnp
from jax.experimental import pallas as pl

_IN_BLOCK = 5000
_OUT_BLOCK = 1000
_SUBS = _IN_BLOCK // _OUT_BLOCK


def _fused_linear_relu(h_ref, w_ref, b_ref, o_ref):
    i = pl.program_id(0)
    sub = jax.lax.rem(i, _SUBS)
    x = h_ref[pl.ds(sub * _OUT_BLOCK, _OUT_BLOCK), :].astype(jnp.bfloat16)
    acc = jax.lax.dot_general(
        x, w_ref[...].astype(jnp.bfloat16), (((1,), (1,)), ((), ())),
        preferred_element_type=jnp.float32)
    o_ref[...] = jnp.maximum(acc + b_ref[...], 0.0)


def kernel(h, edge_index, W, b):
    del edge_index
    n, d_in = h.shape
    d_out = W.shape[0]
    b2 = b.reshape(1, d_out)
    return pl.pallas_call(
        _fused_linear_relu,
        grid=(n // _OUT_BLOCK,),
        in_specs=[
            pl.BlockSpec((_IN_BLOCK, d_in), lambda i: (i // _SUBS, 0),
                         pipeline_mode=pl.Buffered(buffer_count=2)),
            pl.BlockSpec((d_out, d_in), lambda i: (0, 0)),
            pl.BlockSpec((1, d_out), lambda i: (0, 0)),
        ],
        out_specs=pl.BlockSpec((_OUT_BLOCK, d_out), lambda i: (i, 0)),
        out_shape=jax.ShapeDtypeStruct((n, d_out), jnp.float32),
    )(h, W, b2)


# speedup vs baseline: 1.3943x; 1.3943x over previous
import jax
import jax.numpy as jnp
from jax.experimental import pallas as pl

_BLOCK_ROWS = 3336


def _fused_linear_relu(h_ref, w_ref, b_ref, o_ref):
    x = h_ref[...].astype(jnp.bfloat16)
    acc = jax.lax.dot_general(
        x, w_ref[...].astype(jnp.bfloat16), (((1,), (1,)), ((), ())),
        preferred_element_type=jnp.float32)
    o_ref[...] = jnp.maximum(acc + b_ref[...], 0.0)


def kernel(h, edge_index, W, b):
    del edge_index
    n, d_in = h.shape
    d_out = W.shape[0]
    b2 = b.reshape(1, d_out)
    return pl.pallas_call(
        _fused_linear_relu,
        grid=(pl.cdiv(n, _BLOCK_ROWS),),
        in_specs=[
            pl.BlockSpec((_BLOCK_ROWS, d_in), lambda i: (i, 0)),
            pl.BlockSpec((d_out, d_in), lambda i: (0, 0)),
            pl.BlockSpec((1, d_out), lambda i: (0, 0)),
        ],
        out_specs=pl.BlockSpec((_BLOCK_ROWS, d_out), lambda i: (i, 0)),
        out_shape=jax.ShapeDtypeStruct((n, d_out), jnp.float32),
    )(h, W, b2)


# speedup vs baseline: 1.8253x; 1.3091x over previous
import jax
import jax.numpy as jnp
from jax.experimental import pallas as pl

_BLOCK_ROWS = 5000


def _fused_linear_relu(h_ref, w_ref, b_ref, o_ref):
    x = h_ref[...].astype(jnp.bfloat16)
    acc = jax.lax.dot_general(
        x, w_ref[...].astype(jnp.bfloat16), (((1,), (1,)), ((), ())),
        preferred_element_type=jnp.float32,
    )
    o_ref[...] = jnp.maximum(acc + b_ref[...], 0.0)


def kernel(h, edge_index, W, b):
    del edge_index
    n, d_in = h.shape
    d_out = W.shape[0]
    b2 = b.reshape(1, d_out)
    return pl.pallas_call(
        _fused_linear_relu,
        grid=(pl.cdiv(n, _BLOCK_ROWS),),
        in_specs=[
            pl.BlockSpec((_BLOCK_ROWS, d_in), lambda i: (i, 0)),
            pl.BlockSpec((d_out, d_in), lambda i: (0, 0)),
            pl.BlockSpec((1, d_out), lambda i: (0, 0)),
        ],
        out_specs=pl.BlockSpec((_BLOCK_ROWS, d_out), lambda i: (i, 0)),
        out_shape=jax.ShapeDtypeStruct((n, d_out), jnp.float32),
    )(h, W, b2)
